# Initial kernel scaffold; baseline (speedup 1.0000x reference)
import jax
import jax.numpy as jnp
from jax.experimental import pallas as pl


def kernel(h, edge_index, e, W_enc, b_enc, W_layers, b_layers, gamma, beta, W_out, b_out):
    raise NotImplementedError("write your pallas kernel here")


# speedup vs baseline: 105.3384x; 105.3384x over previous
import functools

import jax
import jax.numpy as jnp
import numpy as np
from jax import lax
from jax.experimental import pallas as pl
from jax.experimental.pallas import tpu as pltpu
from jax.experimental.pallas import tpu_sc as plsc

N = 10000
E = 320000
D = 128
H = 128
C = 16
L = 4

NC = 2
NS = 16
NW = NC * NS
B = 128
CH = 79
EPW = CH * B
NB = 10240
STR = NB // NS

_MESH = plsc.VectorSubcoreMesh(core_axis_name="c", subcore_axis_name="s")


@functools.partial(
    pl.kernel,
    out_type=jax.ShapeDtypeStruct((NC, NB, D), jnp.float32),
    mesh=_MESH,
    scratch_types=[
        pltpu.VMEM((B,), jnp.int32),
        pltpu.VMEM((B,), jnp.int32),
        pltpu.VMEM((B, D), jnp.float32),
        pltpu.VMEM_SHARED((NB, D), jnp.float32),
        pltpu.SemaphoreType.DMA,
    ],
)
def _sc_agg(h_hbm, src_hbm, dst_hbm, zero_hbm, out_hbm, src_v, dst_v, rows_v,
            agg_sh, sem):
    c = lax.axis_index("c")
    s = lax.axis_index("s")
    wid = c * jnp.int32(NS) + s
    srow = s * jnp.int32(STR)
    pltpu.sync_copy(zero_hbm, agg_sh.at[pl.ds(srow, STR)])
    plsc.subcore_barrier()
    base = wid * jnp.int32(EPW)

    def body(i, carry):
        off = base + i * jnp.int32(B)
        pltpu.sync_copy(src_hbm.at[pl.ds(off, B)], src_v)
        pltpu.sync_copy(dst_hbm.at[pl.ds(off, B)], dst_v)
        pltpu.async_copy(h_hbm.at[src_v], rows_v, sem).wait()
        pltpu.sync_copy(rows_v, agg_sh.at[dst_v], add=True)
        return carry

    lax.fori_loop(jnp.int32(0), jnp.int32(CH), body, jnp.int32(0))
    plsc.subcore_barrier()
    pltpu.sync_copy(agg_sh.at[pl.ds(srow, STR)],
                    out_hbm.at[c, pl.ds(srow, STR)])


@functools.partial(
    pl.kernel,
    out_type=jax.ShapeDtypeStruct((NC, NB, D), jnp.float32),
    mesh=_MESH,
    scratch_types=[
        pltpu.VMEM((B,), jnp.int32),
        pltpu.VMEM((B, D), jnp.float32),
        pltpu.VMEM_SHARED((NB, D), jnp.float32),
    ],
)
def _sc_deg(ones_hbm, dst_hbm, zero_hbm, out_hbm, dst_v, ones_v, acc_sh):
    c = lax.axis_index("c")
    s = lax.axis_index("s")
    wid = c * jnp.int32(NS) + s
    srow = s * jnp.int32(STR)
    pltpu.sync_copy(zero_hbm, acc_sh.at[pl.ds(srow, STR)])
    pltpu.sync_copy(ones_hbm, ones_v)
    plsc.subcore_barrier()
    base = wid * jnp.int32(EPW)

    def body(i, carry):
        off = base + i * jnp.int32(B)
        pltpu.sync_copy(dst_hbm.at[pl.ds(off, B)], dst_v)
        pltpu.sync_copy(ones_v, acc_sh.at[dst_v], add=True)
        return carry

    lax.fori_loop(jnp.int32(0), jnp.int32(CH), body, jnp.int32(0))
    plsc.subcore_barrier()
    pltpu.sync_copy(acc_sh.at[pl.ds(srow, STR)],
                    out_hbm.at[c, pl.ds(srow, STR)])


_Z = np.int32(0)
R = 2000
G = N // R


def _dot(a, b):
    return jnp.dot(a, b, preferred_element_type=jnp.float32,
                   precision=lax.Precision.HIGHEST)


def _tc_encode(h_ref, w_ref, b_ref, out_ref):
    out_ref[...] = _dot(h_ref[...], w_ref[...]) + b_ref[...]


def _tc_mm1(h_ref, parts_ref, degp_ref, wt_ref, wb_ref, b_ref,
            hl_ref, degs_ref, stats_ref):
    degs = jnp.maximum(degp_ref[0] + degp_ref[1], 1.0)
    degs_ref[...] = degs
    agg = (parts_ref[0] + parts_ref[1]) * (1.0 / degs)
    hl = _dot(h_ref[...], wt_ref[...]) + _dot(agg, wb_ref[...]) + b_ref[...]
    hl_ref[...] = hl

    @pl.when(pl.program_id(0) == 0)
    def _():
        stats_ref[...] = jnp.zeros_like(stats_ref)

    stats_ref[0:1, :] += jnp.sum(hl, axis=0, keepdims=True)
    stats_ref[1:2, :] += jnp.sum(hl * hl, axis=0, keepdims=True)


def _tc_mm(h_ref, parts_ref, degs_ref, wt_ref, wb_ref, b_ref,
           hl_ref, stats_ref):
    agg = (parts_ref[0] + parts_ref[1]) * (1.0 / degs_ref[...])
    hl = _dot(h_ref[...], wt_ref[...]) + _dot(agg, wb_ref[...]) + b_ref[...]
    hl_ref[...] = hl

    @pl.when(pl.program_id(0) == 0)
    def _():
        stats_ref[...] = jnp.zeros_like(stats_ref)

    stats_ref[0:1, :] += jnp.sum(hl, axis=0, keepdims=True)
    stats_ref[1:2, :] += jnp.sum(hl * hl, axis=0, keepdims=True)


def _bn(hl, stats, g, be):
    mu = stats[0:1, :] * (1.0 / N)
    var = stats[1:2, :] * (1.0 / N) - mu * mu
    return (hl - mu) * lax.rsqrt(var + 1e-5) * g + be


def _tc_norm(hl_ref, stats_ref, degs_ref, g_ref, be_ref, out_ref):
    hn = _bn(hl_ref[...], stats_ref[...], g_ref[...], be_ref[...])
    out_ref[...] = jnp.maximum(hn, 0.0) * lax.rsqrt(degs_ref[...])


def _tc_norm_out(hl_ref, stats_ref, degs_ref, g_ref, be_ref, wo_ref, bo_ref,
                 out_ref):
    hn = _bn(hl_ref[...], stats_ref[...], g_ref[...], be_ref[...])
    hr = jnp.maximum(hn, 0.0) * lax.rsqrt(degs_ref[...])
    out_ref[...] = _dot(hr, wo_ref[...]) + bo_ref[...]


def kernel(h, edge_index, e, W_enc, b_enc, W_layers, b_layers, gamma, beta,
           W_out, b_out):
    f32 = jnp.float32
    epn = E // NW
    src = edge_index[0].astype(jnp.int32).reshape(NW, epn)
    dst = edge_index[1].astype(jnp.int32).reshape(NW, epn)
    srcp = jnp.pad(src, ((0, 0), (0, EPW - epn))).reshape(-1)
    dstp = jnp.pad(dst, ((0, 0), (0, EPW - epn)),
                   constant_values=N).reshape(-1)
    zero = jnp.zeros((STR, D), f32)
    ones = jnp.ones((B, D), f32)

    rows = pl.BlockSpec((R, H), lambda i: (i, _Z))
    rows1 = pl.BlockSpec((R, 1), lambda i: (i, _Z))
    prows = pl.BlockSpec((2, R, H), lambda i: (_Z, i, _Z))
    dprows = pl.BlockSpec((2, R, 1), lambda i: (_Z, i, _Z))
    const = pl.BlockSpec((H, H), lambda i: (_Z, _Z))
    cvec = pl.BlockSpec((1, H), lambda i: (_Z, _Z))
    cstat = pl.BlockSpec((8, H), lambda i: (_Z, _Z))
    cout = pl.BlockSpec((H, C), lambda i: (_Z, _Z))
    cbo = pl.BlockSpec((1, C), lambda i: (_Z, _Z))

    h = h.astype(f32)
    h1 = pl.pallas_call(
        _tc_encode,
        grid=(G,),
        in_specs=[rows, const, cvec],
        out_specs=rows,
        out_shape=jax.ShapeDtypeStruct((N, H), f32),
    )(h, W_enc.astype(f32), b_enc.astype(f32).reshape(1, H))

    degp = _sc_deg(ones, dstp, zero)[:, :N, 0:1]

    wts = [(W_layers[i, :H, :].astype(f32), W_layers[i, H:, :].astype(f32),
            b_layers[i].astype(f32).reshape(1, H),
            gamma[i].astype(f32).reshape(1, H),
            beta[i].astype(f32).reshape(1, H)) for i in range(L)]

    parts = _sc_agg(h1, srcp, dstp, zero)
    wt, wb, b, g, be = wts[0]
    hl, degs, stats = pl.pallas_call(
        _tc_mm1,
        grid=(G,),
        in_specs=[rows, prows, dprows, const, const, cvec],
        out_specs=(rows, rows1, cstat),
        out_shape=(jax.ShapeDtypeStruct((N, H), f32),
                   jax.ShapeDtypeStruct((N, 1), f32),
                   jax.ShapeDtypeStruct((8, H), f32)),
    )(h1, parts, degp, wt, wb, b)
    hcur = pl.pallas_call(
        _tc_norm,
        grid=(G,),
        in_specs=[rows, cstat, rows1, cvec, cvec],
        out_specs=rows,
        out_shape=jax.ShapeDtypeStruct((N, H), f32),
    )(hl, stats, degs, g, be)

    for i in range(1, L):
        parts = _sc_agg(hcur, srcp, dstp, zero)
        wt, wb, b, g, be = wts[i]
        hl, stats = pl.pallas_call(
            _tc_mm,
            grid=(G,),
            in_specs=[rows, prows, rows1, const, const, cvec],
            out_specs=(rows, cstat),
            out_shape=(jax.ShapeDtypeStruct((N, H), f32),
                       jax.ShapeDtypeStruct((8, H), f32)),
        )(hcur, parts, degs, wt, wb, b)
        if i < L - 1:
            hcur = pl.pallas_call(
                _tc_norm,
                grid=(G,),
                in_specs=[rows, cstat, rows1, cvec, cvec],
                out_specs=rows,
                out_shape=jax.ShapeDtypeStruct((N, H), f32),
            )(hl, stats, degs, g, be)
        else:
            out = pl.pallas_call(
                _tc_norm_out,
                grid=(G,),
                in_specs=[rows, cstat, rows1, cvec, cvec, cout, cbo],
                out_specs=pl.BlockSpec((R, C), lambda i: (i, _Z)),
                out_shape=jax.ShapeDtypeStruct((N, C), f32),
            )(hl, stats, degs, g, be,
              W_out.astype(f32), b_out.astype(f32).reshape(1, C))
    return out.astype(jnp.float64)
